# baseline (speedup 1.0000x reference)
import functools

import jax
import jax.numpy as jnp
from jax import lax
from jax.experimental import pallas as pl
from jax.experimental.pallas import tpu as pltpu
from jax.experimental.pallas import tpu_sc as plsc

N = 10000
E = 320000
D = 128
H = 128
C = 64
G = 128
EPS = 1e-5

NW = 32
EPW = E // NW
CH = 80
NCHUNK = EPW // CH
ROWA = 624
K = 5
NGRP = NCHUNK // K

_mesh = plsc.VectorSubcoreMesh(core_axis_name="c", subcore_axis_name="s")


def _deg_body(dst_hbm, deg_out, dst_v, deg_v):
    c = lax.axis_index("c")
    s = lax.axis_index("s")
    w = s * 2 + c
    pltpu.sync_copy(dst_hbm.at[w], dst_v)

    def zero(i, _):
        deg_v[pl.ds(i * 16, 16)] = jnp.zeros((16,), jnp.float32)
        return _

    lax.fori_loop(0, N // 16, zero, None)

    ones = jnp.ones((16,), jnp.float32)

    def count(i, _):
        idx = dst_v[pl.ds(i * 16, 16)]
        plsc.addupdate_scatter(deg_v, [idx], ones)
        return _

    lax.fori_loop(0, EPW // 16, count, None)
    pltpu.sync_copy(deg_v, deg_out.at[w])


_deg_kernel = pl.kernel(
    _deg_body,
    out_type=jax.ShapeDtypeStruct((NW, N), jnp.float32),
    mesh=_mesh,
    scratch_types=[
        pltpu.VMEM((EPW,), jnp.int32),
        pltpu.VMEM((N,), jnp.float32),
    ],
    compiler_params=pltpu.CompilerParams(needs_layout_passes=False, use_tc_tiling_on_sc=False),
)


HH = H // 2


def _agg_body(hp_hbm, srcs_hbm, dsts_hbm, out_hbm,
              src_v, dst_v, rows0, rows1, zbuf, acc, semga, semgb, sems):
    c = lax.axis_index("c")
    s = lax.axis_index("s")
    w = s * 2 + c
    pltpu.sync_copy(srcs_hbm.at[w], src_v)
    pltpu.sync_copy(dsts_hbm.at[w], dst_v)

    def zrow(i, _):
        r = i // (HH // 16)
        k = i - r * (HH // 16)
        zbuf[r, pl.ds(k * 16, 16)] = jnp.zeros((16,), jnp.float32)
        return _

    lax.fori_loop(0, CH * (HH // 16), zrow, None)

    for half in range(2):
        hp = hp_hbm.at[half]

        def issue_g(g, bufs, sem):
            for j in range(K):
                pltpu.async_copy(hp.at[src_v.at[g * K + j]], bufs.at[j], sem)

        def wait_g(g, bufs, sem):
            for j in range(K):
                pltpu.make_async_copy(hp.at[src_v.at[g * K + j]], bufs.at[j],
                                      sem).wait()

        def issue_s(g, bufs):
            for j in range(K):
                pltpu.async_copy(bufs.at[j], acc.at[dst_v.at[g * K + j]],
                                 sems, add=True)

        def wait_s(g, bufs):
            for j in range(K):
                pltpu.make_async_copy(bufs.at[j], acc.at[dst_v.at[g * K + j]],
                                      sems).wait()

        issue_g(0, rows0, semga)
        issue_g(1, rows1, semgb)

        def zcopy(j, _):
            pltpu.sync_copy(zbuf, acc.at[pl.ds(s * ROWA + j * CH, CH)])
            return _

        lax.fori_loop(0, ROWA // CH, zcopy, None)

        @pl.when(s < 15)
        def _():
            pltpu.sync_copy(zbuf.at[pl.ds(0, ROWA % CH)],
                            acc.at[pl.ds(s * ROWA + (ROWA // CH) * CH,
                                         ROWA % CH)])

        R15 = N - 15 * ROWA

        @pl.when(s == 15)
        def _():
            for j in range(ROWA // CH, R15 // CH):
                pltpu.sync_copy(zbuf, acc.at[pl.ds(15 * ROWA + j * CH, CH)])
            if R15 % CH:
                pltpu.sync_copy(zbuf.at[pl.ds(0, R15 % CH)],
                                acc.at[pl.ds(15 * ROWA + (R15 // CH) * CH,
                                             R15 % CH)])

        plsc.subcore_barrier()

        wait_g(0, rows0, semga)
        issue_s(0, rows0)

        def step(p, _):
            ga = 2 * p + 1
            wait_g(ga, rows1, semgb)
            wait_s(ga - 1, rows0)
            issue_s(ga, rows1)
            issue_g(ga + 1, rows0, semga)
            gb = 2 * p + 2
            wait_g(gb, rows0, semga)
            wait_s(gb - 1, rows1)
            issue_s(gb, rows0)
            issue_g(gb + 1, rows1, semgb)
            return _

        lax.fori_loop(0, (NGRP - 3) // 2, step, None)
        wait_g(NGRP - 2, rows1, semgb)
        wait_s(NGRP - 3, rows0)
        issue_s(NGRP - 2, rows1)
        issue_g(NGRP - 1, rows0, semga)
        wait_g(NGRP - 1, rows0, semga)
        wait_s(NGRP - 2, rows1)
        issue_s(NGRP - 1, rows0)
        wait_s(NGRP - 1, rows0)

        plsc.subcore_barrier()

        @pl.when(s < 15)
        def _():
            pltpu.sync_copy(acc.at[pl.ds(s * ROWA, ROWA)],
                            out_hbm.at[c, half, pl.ds(s * ROWA, ROWA)])

        @pl.when(s == 15)
        def _():
            pltpu.sync_copy(acc.at[pl.ds(15 * ROWA, N - 15 * ROWA)],
                            out_hbm.at[c, half, pl.ds(15 * ROWA,
                                                      N - 15 * ROWA)])

        plsc.subcore_barrier()


_agg_kernel = pl.kernel(
    _agg_body,
    out_type=jax.ShapeDtypeStruct((2, 2, N, HH), jnp.float32),
    mesh=_mesh,
    scratch_types=[
        pltpu.VMEM((NCHUNK, CH), jnp.int32),
        pltpu.VMEM((NCHUNK, CH), jnp.int32),
        pltpu.VMEM((K, CH, HH), jnp.float32),
        pltpu.VMEM((K, CH, HH), jnp.float32),
        pltpu.VMEM((CH, HH), jnp.float32),
        pltpu.VMEM_SHARED((N, HH), jnp.float32),
        pltpu.SemaphoreType.DMA,
        pltpu.SemaphoreType.DMA,
        pltpu.SemaphoreType.DMA,
    ],
    compiler_params=pltpu.CompilerParams(needs_layout_passes=False, use_tc_tiling_on_sc=False),
)


RB = 10000
NRB = N // RB


def _dis(degT_blk):
    deg = jnp.sum(degT_blk, axis=1, keepdims=True) + 1.0
    return lax.rsqrt(deg)


def _tc1_body(degT_ref, x_ref, w1_ref, hp1_ref):
    dis = _dis(degT_ref[...])
    h = jnp.dot(x_ref[...], w1_ref[...], preferred_element_type=jnp.float32)
    hp = h * dis
    hp1_ref[0] = hp[:, :HH]
    hp1_ref[1] = hp[:, HH:]


def _ln_relu(z, g_ref, be_ref):
    m = jnp.mean(z, axis=-1, keepdims=True)
    v = jnp.mean((z - m) ** 2, axis=-1, keepdims=True)
    zn = (z - m) * lax.rsqrt(v + EPS) * g_ref[...] + be_ref[...]
    return jnp.maximum(zn, 0.0)


def _combine(acc_ref, hp_ref, dis, b_ref, g_ref, be_ref):
    zl = dis * (acc_ref[0, 0] + acc_ref[1, 0] + hp_ref[0]) + b_ref[:, :HH]
    zr = dis * (acc_ref[0, 1] + acc_ref[1, 1] + hp_ref[1]) + b_ref[:, HH:]
    z = jnp.concatenate([zl, zr], axis=-1)
    return _ln_relu(z, g_ref, be_ref)


def _tc2_body(acc_ref, hp1_ref, degT_ref, b1_ref, g1_ref, be1_ref, w2_ref,
              hp2_ref):
    dis = _dis(degT_ref[...])
    h = _combine(acc_ref, hp1_ref, dis, b1_ref, g1_ref, be1_ref)
    hp2 = jnp.dot(h, w2_ref[...], preferred_element_type=jnp.float32) * dis
    hp2_ref[0] = hp2[:, :HH]
    hp2_ref[1] = hp2[:, HH:]


def _tc3_body(acc_ref, hp2_ref, degT_ref, b2_ref, g2_ref, be2_ref,
              batch_ref, wl_ref, bl_ref, sums_ref, cnts_ref, out_ref):
    i = pl.program_id(0)
    dis = _dis(degT_ref[...])
    h = _combine(acc_ref, hp2_ref, dis, b2_ref, g2_ref, be2_ref)

    b = batch_ref[0, 0, :]
    oh = (b[:, None] == lax.broadcasted_iota(jnp.int32, (RB, G), 1)).astype(
        jnp.float32)
    dn = (((0,), (0,)), ((), ()))
    sums_p = lax.dot_general(oh, h, dn, preferred_element_type=jnp.float32)
    cnts_p = lax.dot_general(oh, jnp.ones_like(h), dn,
                             preferred_element_type=jnp.float32)

    @pl.when(i == 0)
    def _():
        sums_ref[...] = sums_p
        cnts_ref[...] = cnts_p

    @pl.when(i > 0)
    def _():
        sums_ref[...] += sums_p
        cnts_ref[...] += cnts_p

    @pl.when(i == NRB - 1)
    def _():
        pooled = sums_ref[...] / jnp.maximum(cnts_ref[...], 1.0)
        out_ref[...] = jnp.dot(pooled, wl_ref[...],
                               preferred_element_type=jnp.float32) + bl_ref[...]


def _full2(shape):
    return pl.BlockSpec(shape, lambda i: (0,) * len(shape))


_tc1 = pl.pallas_call(
    _tc1_body,
    grid=(NRB,),
    in_specs=[
        pl.BlockSpec((RB, NW), lambda i: (i, 0)),
        pl.BlockSpec((RB, D), lambda i: (i, 0)),
        _full2((D, H)),
    ],
    out_specs=pl.BlockSpec((2, RB, HH), lambda i: (0, i, 0)),
    out_shape=jax.ShapeDtypeStruct((2, N, HH), jnp.float32),
)

_tc2 = pl.pallas_call(
    _tc2_body,
    grid=(NRB,),
    in_specs=[
        pl.BlockSpec((2, 2, RB, HH), lambda i: (0, 0, i, 0)),
        pl.BlockSpec((2, RB, HH), lambda i: (0, i, 0)),
        pl.BlockSpec((RB, NW), lambda i: (i, 0)),
        _full2((1, H)), _full2((1, H)), _full2((1, H)),
        _full2((H, H)),
    ],
    out_specs=pl.BlockSpec((2, RB, HH), lambda i: (0, i, 0)),
    out_shape=jax.ShapeDtypeStruct((2, N, HH), jnp.float32),
)

_tc3 = pl.pallas_call(
    _tc3_body,
    grid=(NRB,),
    in_specs=[
        pl.BlockSpec((2, 2, RB, HH), lambda i: (0, 0, i, 0)),
        pl.BlockSpec((2, RB, HH), lambda i: (0, i, 0)),
        pl.BlockSpec((RB, NW), lambda i: (i, 0)),
        _full2((1, H)), _full2((1, H)), _full2((1, H)),
        pl.BlockSpec((1, 1, RB), lambda i: (i, 0, 0)),
        _full2((H, C)), _full2((1, C)),
    ],
    out_specs=[_full2((G, H)), _full2((G, H)), _full2((G, C))],
    out_shape=[
        jax.ShapeDtypeStruct((G, H), jnp.float32),
        jax.ShapeDtypeStruct((G, H), jnp.float32),
        jax.ShapeDtypeStruct((G, C), jnp.float32),
    ],
)


@jax.jit
def kernel(x, edge_index, batch, W1, b1, g1, be1, W2, b2, g2, be2, Wl, bl):
    src = edge_index[0].astype(jnp.int32)
    dst = edge_index[1].astype(jnp.int32)
    srcs3 = src.reshape(NW, NCHUNK, CH)
    dsts3 = dst.reshape(NW, NCHUNK, CH)
    dst2 = dst.reshape(NW, EPW)
    batch3 = batch.astype(jnp.int32).reshape(NRB, 1, RB)

    deg_parts = _deg_kernel(dst2)
    degT = deg_parts.T

    hp1 = _tc1(degT, x, W1)
    acc1 = _agg_kernel(hp1, srcs3, dsts3)
    hp2 = _tc2(acc1, hp1, degT, b1.reshape(1, H), g1.reshape(1, H),
               be1.reshape(1, H), W2)
    acc2 = _agg_kernel(hp2, srcs3, dsts3)
    _, _, out = _tc3(acc2, hp2, degT, b2.reshape(1, H), g2.reshape(1, H),
                     be2.reshape(1, H), batch3, Wl, bl.reshape(1, C))
    return out


# speedup vs baseline: 1.0275x; 1.0275x over previous
import functools

import jax
import jax.numpy as jnp
from jax import lax
from jax.experimental import pallas as pl
from jax.experimental.pallas import tpu as pltpu
from jax.experimental.pallas import tpu_sc as plsc

N = 10000
E = 320000
D = 128
H = 128
C = 64
G = 128
EPS = 1e-5

NW = 32
EPW = E // NW
CH = 80
NCHUNK = EPW // CH
ROWA = 624
K = 5
NGRP = NCHUNK // K

_mesh = plsc.VectorSubcoreMesh(core_axis_name="c", subcore_axis_name="s")


def _deg_body(dst_hbm, deg_out, dst_v, deg_v):
    c = lax.axis_index("c")
    s = lax.axis_index("s")
    w = s * 2 + c
    pltpu.sync_copy(dst_hbm.at[w], dst_v)

    def zero(i, _):
        deg_v[pl.ds(i * 16, 16)] = jnp.zeros((16,), jnp.float32)
        return _

    lax.fori_loop(0, N // 16, zero, None)

    ones = jnp.ones((16,), jnp.float32)

    def count(i, _):
        idx = dst_v[pl.ds(i * 16, 16)]
        plsc.addupdate_scatter(deg_v, [idx], ones)
        return _

    lax.fori_loop(0, EPW // 16, count, None)
    pltpu.sync_copy(deg_v, deg_out.at[w])


_deg_kernel = pl.kernel(
    _deg_body,
    out_type=jax.ShapeDtypeStruct((NW, N), jnp.float32),
    mesh=_mesh,
    scratch_types=[
        pltpu.VMEM((EPW,), jnp.int32),
        pltpu.VMEM((N,), jnp.float32),
    ],
    compiler_params=pltpu.CompilerParams(needs_layout_passes=False, use_tc_tiling_on_sc=False),
)


HH = H // 2


def _agg_body(hp_hbm, srcs_hbm, dsts_hbm, out_hbm,
              src_v, dst_v, rows0, rows1, zbuf, acc, semga, semgb, sems):
    c = lax.axis_index("c")
    s = lax.axis_index("s")
    w = s * 2 + c
    pltpu.sync_copy(srcs_hbm.at[w], src_v)
    pltpu.sync_copy(dsts_hbm.at[w], dst_v)

    def zrow(i, _):
        r = i // (HH // 16)
        k = i - r * (HH // 16)
        zbuf[r, pl.ds(k * 16, 16)] = jnp.zeros((16,), jnp.float32)
        return _

    lax.fori_loop(0, CH * (HH // 16), zrow, None)

    def copy_out(half):
        @pl.when(s < 15)
        def _():
            pltpu.sync_copy(acc.at[pl.ds(s * ROWA, ROWA)],
                            out_hbm.at[c, half, pl.ds(s * ROWA, ROWA)])

        @pl.when(s == 15)
        def _():
            pltpu.sync_copy(acc.at[pl.ds(15 * ROWA, N - 15 * ROWA)],
                            out_hbm.at[c, half, pl.ds(15 * ROWA,
                                                      N - 15 * ROWA)])

        plsc.subcore_barrier()

    for half in range(2):
        hp = hp_hbm.at[half]

        def issue_g(g, bufs, sem):
            for j in range(K):
                pltpu.async_copy(hp.at[src_v.at[g * K + j]], bufs.at[j], sem)

        def wait_g(g, bufs, sem):
            for j in range(K):
                pltpu.make_async_copy(hp.at[src_v.at[g * K + j]], bufs.at[j],
                                      sem).wait()

        def issue_s(g, bufs):
            for j in range(K):
                pltpu.async_copy(bufs.at[j], acc.at[dst_v.at[g * K + j]],
                                 sems, add=True)

        def wait_s(g, bufs):
            for j in range(K):
                pltpu.make_async_copy(bufs.at[j], acc.at[dst_v.at[g * K + j]],
                                      sems).wait()

        issue_g(0, rows0, semga)
        issue_g(1, rows1, semgb)

        if half > 0:
            copy_out(half - 1)

        def zcopy(j, _):
            pltpu.sync_copy(zbuf, acc.at[pl.ds(s * ROWA + j * CH, CH)])
            return _

        lax.fori_loop(0, ROWA // CH, zcopy, None)

        @pl.when(s < 15)
        def _():
            pltpu.sync_copy(zbuf.at[pl.ds(0, ROWA % CH)],
                            acc.at[pl.ds(s * ROWA + (ROWA // CH) * CH,
                                         ROWA % CH)])

        R15 = N - 15 * ROWA

        @pl.when(s == 15)
        def _():
            for j in range(ROWA // CH, R15 // CH):
                pltpu.sync_copy(zbuf, acc.at[pl.ds(15 * ROWA + j * CH, CH)])
            if R15 % CH:
                pltpu.sync_copy(zbuf.at[pl.ds(0, R15 % CH)],
                                acc.at[pl.ds(15 * ROWA + (R15 // CH) * CH,
                                             R15 % CH)])

        plsc.subcore_barrier()

        wait_g(0, rows0, semga)
        issue_s(0, rows0)

        def step(p, _):
            ga = 2 * p + 1
            wait_g(ga, rows1, semgb)
            wait_s(ga - 1, rows0)
            issue_s(ga, rows1)
            issue_g(ga + 1, rows0, semga)
            gb = 2 * p + 2
            wait_g(gb, rows0, semga)
            wait_s(gb - 1, rows1)
            issue_s(gb, rows0)
            issue_g(gb + 1, rows1, semgb)
            return _

        lax.fori_loop(0, (NGRP - 3) // 2, step, None)
        wait_g(NGRP - 2, rows1, semgb)
        wait_s(NGRP - 3, rows0)
        issue_s(NGRP - 2, rows1)
        issue_g(NGRP - 1, rows0, semga)
        wait_g(NGRP - 1, rows0, semga)
        wait_s(NGRP - 2, rows1)
        issue_s(NGRP - 1, rows0)
        wait_s(NGRP - 1, rows0)

        plsc.subcore_barrier()

    copy_out(1)


_agg_kernel = pl.kernel(
    _agg_body,
    out_type=jax.ShapeDtypeStruct((2, 2, N, HH), jnp.float32),
    mesh=_mesh,
    scratch_types=[
        pltpu.VMEM((NCHUNK, CH), jnp.int32),
        pltpu.VMEM((NCHUNK, CH), jnp.int32),
        pltpu.VMEM((K, CH, HH), jnp.float32),
        pltpu.VMEM((K, CH, HH), jnp.float32),
        pltpu.VMEM((CH, HH), jnp.float32),
        pltpu.VMEM_SHARED((N, HH), jnp.float32),
        pltpu.SemaphoreType.DMA,
        pltpu.SemaphoreType.DMA,
        pltpu.SemaphoreType.DMA,
    ],
    compiler_params=pltpu.CompilerParams(needs_layout_passes=False, use_tc_tiling_on_sc=False),
)


RB = 2000
NRB = N // RB


def _dis(degT_blk):
    deg = jnp.sum(degT_blk, axis=1, keepdims=True) + 1.0
    return lax.rsqrt(deg)


def _tc1_body(degT_ref, x_ref, w1_ref, hp1_ref):
    dis = _dis(degT_ref[...])
    h = jnp.dot(x_ref[...], w1_ref[...], preferred_element_type=jnp.float32)
    hp = h * dis
    hp1_ref[0] = hp[:, :HH]
    hp1_ref[1] = hp[:, HH:]


def _ln_relu(z, g_ref, be_ref):
    m = jnp.mean(z, axis=-1, keepdims=True)
    v = jnp.mean((z - m) ** 2, axis=-1, keepdims=True)
    zn = (z - m) * lax.rsqrt(v + EPS) * g_ref[...] + be_ref[...]
    return jnp.maximum(zn, 0.0)


def _combine(acc_ref, hp_ref, dis, b_ref, g_ref, be_ref):
    zl = dis * (acc_ref[0, 0] + acc_ref[1, 0] + hp_ref[0]) + b_ref[:, :HH]
    zr = dis * (acc_ref[0, 1] + acc_ref[1, 1] + hp_ref[1]) + b_ref[:, HH:]
    z = jnp.concatenate([zl, zr], axis=-1)
    return _ln_relu(z, g_ref, be_ref)


def _tc2_body(acc_ref, hp1_ref, degT_ref, b1_ref, g1_ref, be1_ref, w2_ref,
              hp2_ref):
    dis = _dis(degT_ref[...])
    h = _combine(acc_ref, hp1_ref, dis, b1_ref, g1_ref, be1_ref)
    hp2 = jnp.dot(h, w2_ref[...], preferred_element_type=jnp.float32) * dis
    hp2_ref[0] = hp2[:, :HH]
    hp2_ref[1] = hp2[:, HH:]


def _tc3_body(acc_ref, hp2_ref, degT_ref, b2_ref, g2_ref, be2_ref,
              batch_ref, wl_ref, bl_ref, sums_ref, cnts_ref, out_ref):
    i = pl.program_id(0)
    dis = _dis(degT_ref[...])
    h = _combine(acc_ref, hp2_ref, dis, b2_ref, g2_ref, be2_ref)

    b = batch_ref[0, 0, :]
    oh = (b[:, None] == lax.broadcasted_iota(jnp.int32, (RB, G), 1)).astype(
        jnp.float32)
    dn = (((0,), (0,)), ((), ()))
    sums_p = lax.dot_general(oh, h, dn, preferred_element_type=jnp.float32)
    cnts_p = lax.dot_general(oh, jnp.ones_like(h), dn,
                             preferred_element_type=jnp.float32)

    @pl.when(i == 0)
    def _():
        sums_ref[...] = sums_p
        cnts_ref[...] = cnts_p

    @pl.when(i > 0)
    def _():
        sums_ref[...] += sums_p
        cnts_ref[...] += cnts_p

    @pl.when(i == NRB - 1)
    def _():
        pooled = sums_ref[...] / jnp.maximum(cnts_ref[...], 1.0)
        out_ref[...] = jnp.dot(pooled, wl_ref[...],
                               preferred_element_type=jnp.float32) + bl_ref[...]


def _full2(shape):
    return pl.BlockSpec(shape, lambda i: (0,) * len(shape))


_tc1 = pl.pallas_call(
    _tc1_body,
    grid=(NRB,),
    in_specs=[
        pl.BlockSpec((RB, NW), lambda i: (i, 0)),
        pl.BlockSpec((RB, D), lambda i: (i, 0)),
        _full2((D, H)),
    ],
    out_specs=pl.BlockSpec((2, RB, HH), lambda i: (0, i, 0)),
    out_shape=jax.ShapeDtypeStruct((2, N, HH), jnp.float32),
)

_tc2 = pl.pallas_call(
    _tc2_body,
    grid=(NRB,),
    in_specs=[
        pl.BlockSpec((2, 2, RB, HH), lambda i: (0, 0, i, 0)),
        pl.BlockSpec((2, RB, HH), lambda i: (0, i, 0)),
        pl.BlockSpec((RB, NW), lambda i: (i, 0)),
        _full2((1, H)), _full2((1, H)), _full2((1, H)),
        _full2((H, H)),
    ],
    out_specs=pl.BlockSpec((2, RB, HH), lambda i: (0, i, 0)),
    out_shape=jax.ShapeDtypeStruct((2, N, HH), jnp.float32),
)

_tc3 = pl.pallas_call(
    _tc3_body,
    grid=(NRB,),
    in_specs=[
        pl.BlockSpec((2, 2, RB, HH), lambda i: (0, 0, i, 0)),
        pl.BlockSpec((2, RB, HH), lambda i: (0, i, 0)),
        pl.BlockSpec((RB, NW), lambda i: (i, 0)),
        _full2((1, H)), _full2((1, H)), _full2((1, H)),
        pl.BlockSpec((1, 1, RB), lambda i: (i, 0, 0)),
        _full2((H, C)), _full2((1, C)),
    ],
    out_specs=[_full2((G, H)), _full2((G, H)), _full2((G, C))],
    out_shape=[
        jax.ShapeDtypeStruct((G, H), jnp.float32),
        jax.ShapeDtypeStruct((G, H), jnp.float32),
        jax.ShapeDtypeStruct((G, C), jnp.float32),
    ],
)


@jax.jit
def kernel(x, edge_index, batch, W1, b1, g1, be1, W2, b2, g2, be2, Wl, bl):
    src = edge_index[0].astype(jnp.int32)
    dst = edge_index[1].astype(jnp.int32)
    srcs3 = src.reshape(NW, NCHUNK, CH)
    dsts3 = dst.reshape(NW, NCHUNK, CH)
    dst2 = dst.reshape(NW, EPW)
    batch3 = batch.astype(jnp.int32).reshape(NRB, 1, RB)

    deg_parts = _deg_kernel(dst2)
    degT = deg_parts.T

    hp1 = _tc1(degT, x, W1)
    acc1 = _agg_kernel(hp1, srcs3, dsts3)
    hp2 = _tc2(acc1, hp1, degT, b1.reshape(1, H), g1.reshape(1, H),
               be1.reshape(1, H), W2)
    acc2 = _agg_kernel(hp2, srcs3, dsts3)
    _, _, out = _tc3(acc2, hp2, degT, b2.reshape(1, H), g2.reshape(1, H),
                     be2.reshape(1, H), batch3, Wl, bl.reshape(1, C))
    return out
